# baseline (speedup 1.0000x reference)
import functools

import jax
import jax.numpy as jnp
from jax import lax
from jax.experimental import pallas as pl
from jax.experimental.pallas import tpu as pltpu
from jax.experimental.pallas import tpu_sc as plsc

_M, _N = 128, 32768
_NW = 32
_ROWS_PER_W = _M // _NW
_L = 16
_U = 8
_H = _N // 2


def _sc_body(x_hbm, out_hbm, ring, ovb, tv, sem_in, sem_out):
    wid = lax.axis_index("s") * 2 + lax.axis_index("c")
    row0 = wid * _ROWS_PER_W
    lanes = lax.iota(jnp.int32, _L)
    chunk = _U * _L
    n_iters = _H // chunk
    zeros8 = tuple(jnp.zeros((_L,), jnp.float32) for _ in range(_U))

    cp_a = pltpu.async_copy(x_hbm.at[row0, pl.ds(0, _H)], ring.at[0], sem_in)
    cp_b = pltpu.async_copy(x_hbm.at[row0, pl.ds(_H, _H)], ring.at[1], sem_in)
    out_cps = ()
    for r in range(_ROWS_PER_W):
        row = row0 + r
        s0, s1 = (2 * r) % 4, (2 * r + 1) % 4
        cp_a.wait()
        cp_b.wait()

        def p1(slot):
            @plsc.parallel_loop(0, n_iters, carry=zeros8)
            def accs(i, accs):
                base = i * chunk
                return tuple(
                    jnp.maximum(a, jnp.abs(ring[slot, pl.ds(base + j * _L, _L)]))
                    for j, a in enumerate(accs)
                )

            return functools.reduce(jnp.maximum, accs)

        acc = jnp.maximum(p1(s0), p1(s1))

        if r + 1 < _ROWS_PER_W:
            n0, n1 = (2 * r + 2) % 4, (2 * r + 3) % 4
            cp_a = pltpu.async_copy(
                x_hbm.at[row + 1, pl.ds(0, _H)], ring.at[n0], sem_in
            )
            cp_b = pltpu.async_copy(
                x_hbm.at[row + 1, pl.ds(_H, _H)], ring.at[n1], sem_in
            )

        for s in (1, 2, 4, 8):
            tv[...] = acc
            acc = jnp.maximum(acc, plsc.load_gather(tv, [lanes ^ s]))
        thr = 0.5 * (acc + 1e-6)
        nthr = -thr
        one = jnp.full((_L,), 1.0, jnp.float32)
        mone = jnp.full((_L,), -1.0, jnp.float32)
        zero = jnp.zeros((_L,), jnp.float32)

        for cp in out_cps:
            cp.wait()

        def p2(slot, out_slot):
            @plsc.parallel_loop(0, n_iters)
            def _(i):
                base = i * chunk
                for j in range(_U):
                    v = ring[slot, pl.ds(base + j * _L, _L)]
                    ovb[out_slot, pl.ds(base + j * _L, _L)] = jnp.where(
                        v > thr, one, jnp.where(v < nthr, mone, zero)
                    )

        p2(s0, 0)
        o_a = pltpu.async_copy(ovb.at[0], out_hbm.at[row, pl.ds(0, _H)], sem_out)
        p2(s1, 1)
        o_b = pltpu.async_copy(ovb.at[1], out_hbm.at[row, pl.ds(_H, _H)], sem_out)
        out_cps = (o_a, o_b)

    for cp in out_cps:
        cp.wait()


def kernel(x):
    f = pl.kernel(
        _sc_body,
        out_type=jax.ShapeDtypeStruct((_M, _N), jnp.float32),
        mesh=plsc.VectorSubcoreMesh(core_axis_name="c", subcore_axis_name="s"),
        compiler_params=pltpu.CompilerParams(needs_layout_passes=False),
        scratch_types=[
            pltpu.VMEM((4, _H), jnp.float32),
            pltpu.VMEM((2, _H), jnp.float32),
            pltpu.VMEM((_L,), jnp.float32),
            pltpu.SemaphoreType.DMA,
            pltpu.SemaphoreType.DMA,
        ],
    )
    return f(x)


# speedup vs baseline: 2.2918x; 2.2918x over previous
import jax
import jax.numpy as jnp
from jax.experimental import pallas as pl


_ROWS_PER_BLOCK = 8


def _quant_block(x_ref, o_ref):
    x = x_ref[...]
    gamma = jnp.max(jnp.abs(x), axis=-1, keepdims=True)
    thr = 0.5 * (gamma + 1e-6)
    o_ref[...] = jnp.where(x > thr, 1.0, jnp.where(x < -thr, -1.0, 0.0))


def kernel(x):
    m, n = x.shape
    grid = (m // _ROWS_PER_BLOCK,)
    return pl.pallas_call(
        _quant_block,
        grid=grid,
        in_specs=[pl.BlockSpec((_ROWS_PER_BLOCK, n), lambda i: (i, 0))],
        out_specs=pl.BlockSpec((_ROWS_PER_BLOCK, n), lambda i: (i, 0)),
        out_shape=jax.ShapeDtypeStruct((m, n), x.dtype),
    )(x)


# speedup vs baseline: 3.1098x; 1.3569x over previous
import jax
import jax.numpy as jnp
from jax.experimental import pallas as pl


_ROWS_PER_BLOCK = 16


def _quant_block(x_ref, o_ref):
    x = x_ref[...]
    gamma = jnp.max(jnp.abs(x), axis=-1, keepdims=True)
    thr = 0.5 * (gamma + 1e-6)
    o_ref[...] = jnp.where(x > thr, 1.0, jnp.where(x < -thr, -1.0, 0.0))


def kernel(x):
    m, n = x.shape
    grid = (m // _ROWS_PER_BLOCK,)
    return pl.pallas_call(
        _quant_block,
        grid=grid,
        in_specs=[pl.BlockSpec((_ROWS_PER_BLOCK, n), lambda i: (i, 0))],
        out_specs=pl.BlockSpec((_ROWS_PER_BLOCK, n), lambda i: (i, 0)),
        out_shape=jax.ShapeDtypeStruct((m, n), x.dtype),
    )(x)


# speedup vs baseline: 3.3686x; 1.0832x over previous
import jax
import jax.numpy as jnp
from jax.experimental import pallas as pl


_ROWS_PER_BLOCK = 32


def _quant_block(x_ref, o_ref):
    x = x_ref[...]
    gamma = jnp.max(jnp.abs(x), axis=-1, keepdims=True)
    thr = 0.5 * (gamma + 1e-6)
    o_ref[...] = jnp.where(x > thr, 1.0, jnp.where(x < -thr, -1.0, 0.0))


def kernel(x):
    m, n = x.shape
    grid = (m // _ROWS_PER_BLOCK,)
    return pl.pallas_call(
        _quant_block,
        grid=grid,
        in_specs=[pl.BlockSpec((_ROWS_PER_BLOCK, n), lambda i: (i, 0))],
        out_specs=pl.BlockSpec((_ROWS_PER_BLOCK, n), lambda i: (i, 0)),
        out_shape=jax.ShapeDtypeStruct((m, n), x.dtype),
    )(x)


# speedup vs baseline: 3.7666x; 1.1182x over previous
import jax
import jax.numpy as jnp
from jax.experimental import pallas as pl


_ROWS_PER_BLOCK = 64


def _quant_block(x_ref, o_ref):
    x = x_ref[...]
    gamma = jnp.max(jnp.abs(x), axis=-1, keepdims=True)
    thr = 0.5 * (gamma + 1e-6)
    o_ref[...] = jnp.where(x > thr, 1.0, jnp.where(x < -thr, -1.0, 0.0))


def kernel(x):
    m, n = x.shape
    grid = (m // _ROWS_PER_BLOCK,)
    return pl.pallas_call(
        _quant_block,
        grid=grid,
        in_specs=[pl.BlockSpec((_ROWS_PER_BLOCK, n), lambda i: (i, 0))],
        out_specs=pl.BlockSpec((_ROWS_PER_BLOCK, n), lambda i: (i, 0)),
        out_shape=jax.ShapeDtypeStruct((m, n), x.dtype),
    )(x)
